# baseline (speedup 1.0000x reference)
import functools

import jax
import jax.numpy as jnp
from jax import lax
from jax.experimental import pallas as pl
from jax.experimental.pallas import tpu as pltpu
from jax.experimental.pallas import tpu_sc as plsc

N_NODES = 10000
N_EDGES = 160000
HIDDEN = 256
HALF = 128
N_RBF = 16

NC = 2
NS = 16
CHUNK = 40
EDGES_PER_SUB = N_EDGES // NS
NCHUNK = EDGES_PER_SUB // CHUNK
G = 10
GP = G + 2
NBLK = NCHUNK // G
NPAIR_BLK = G // 2
N_PAD = 10112
ROWS_PER_SUB = N_PAD // NS
E_PAD = (NS - 1) * EDGES_PER_SUB + (NCHUNK + 2) * CHUNK


def _silu(x):
    return x * jax.nn.sigmoid(x)


_sc_mesh = plsc.VectorSubcoreMesh(core_axis_name="c", subcore_axis_name="s")


@functools.partial(
    pl.kernel,
    out_type=jax.ShapeDtypeStruct((NC, N_PAD, HALF), jnp.float32),
    scratch_types=[
        pltpu.VMEM((GP, CHUNK), jnp.int32),
        pltpu.VMEM((GP, CHUNK), jnp.int32),
        pltpu.VMEM((G, CHUNK), jnp.int32),
        pltpu.VMEM((G, CHUNK), jnp.int32),
        pltpu.VMEM((CHUNK, HALF), jnp.float32),
        pltpu.VMEM((CHUNK, HALF), jnp.float32),
        pltpu.VMEM((CHUNK, HALF // 2), jnp.int32),
        pltpu.VMEM((CHUNK, HALF // 2), jnp.int32),
        pltpu.VMEM((CHUNK, HALF), jnp.float32),
        pltpu.VMEM((CHUNK, HALF), jnp.float32),
        pltpu.VMEM_SHARED((N_PAD, HALF), jnp.float32),
        pltpu.SemaphoreType.DMA,
        pltpu.SemaphoreType.DMA,
        pltpu.SemaphoreType.DMA,
        pltpu.SemaphoreType.DMA,
        pltpu.SemaphoreType.DMA,
        pltpu.SemaphoreType.DMA,
    ],
    mesh=_sc_mesh,
)
def _sc_fused(hcat_hbm, gate_hbm, src_hbm, dst_hbm, zeros_hbm, out_hbm,
              isrcA, isrcB, idstA, idstB, hbuf0, hbuf1, gbuf0, gbuf1, mbuf0,
              mbuf1, aggr_sh, sh0, sh1, sg0, sg1, ss0, ss1):
    c = lax.axis_index("c")
    s = lax.axis_index("s")
    pltpu.sync_copy(zeros_hbm, aggr_sh.at[pl.ds(s * ROWS_PER_SUB, ROWS_PER_SUB)])
    plsc.subcore_barrier()
    ebase = s * EDGES_PER_SUB
    staging = [(isrcA, idstA), (isrcB, idstB)]

    def issue(gbase, isrc_g, r, hbuf, gbuf, sh, sg):
        pltpu.async_copy(hcat_hbm.at[isrc_g.at[r]], hbuf, sh)
        pltpu.async_copy(gate_hbm.at[c, pl.ds(ebase + (gbase + r) * CHUNK, CHUNK)],
                         gbuf, sg)

    def wait(gbase, isrc_g, r, hbuf, gbuf, sh, sg):
        pltpu.make_async_copy(hcat_hbm.at[isrc_g.at[r]], hbuf, sh).wait()
        pltpu.make_async_copy(
            gate_hbm.at[c, pl.ds(ebase + (gbase + r) * CHUNK, CHUNK)],
            gbuf, sg).wait()

    def wait_ss(mbuf, sem):
        pltpu.make_async_copy(mbuf, aggr_sh.at[pl.ds(0, CHUNK)], sem).wait()

    def mul(hbuf, gbuf, mbuf):
        def row(r, carry):
            for k in range(HALF // 32):
                g32 = gbuf[r, pl.ds(k * 16, 16)]
                lo = (g32 & 0xFFFF).astype(jnp.float32)
                hi = lax.shift_right_logical(g32, 16).astype(jnp.float32)
                sl0 = pl.ds(k * 32, 16)
                sl1 = pl.ds(k * 32 + 16, 16)
                mbuf[r, sl0] = lo * hbuf[r, sl0]
                mbuf[r, sl1] = hi * hbuf[r, sl1]
            return carry
        lax.fori_loop(0, CHUNK, row, 0)

    pltpu.sync_copy(src_hbm.at[c, s, 0], isrcA)
    pltpu.sync_copy(dst_hbm.at[s, 0], idstA)
    issue(0, isrcA, 0, hbuf0, gbuf0, sh0, sg0)
    issue(0, isrcA, 1, hbuf1, gbuf1, sh1, sg1)

    for g in range(NBLK):
        isrc_g, idst_g = staging[g % 2]
        if g > 0:
            pltpu.sync_copy(src_hbm.at[c, s, g], isrc_g)
            pltpu.sync_copy(dst_hbm.at[s, g], idst_g)
        gbase = g * G
        first_block = g == 0

        def pair(t, carry, isrc_g=isrc_g, idst_g=idst_g, gbase=gbase,
                 first_block=first_block):
            r0 = 2 * t
            r1 = r0 + 1
            wait(gbase, isrc_g, r0, hbuf0, gbuf0, sh0, sg0)
            if first_block:
                @pl.when(t > 0)
                def _():
                    wait_ss(mbuf0, ss0)
            else:
                wait_ss(mbuf0, ss0)
            mul(hbuf0, gbuf0, mbuf0)
            pltpu.async_copy(mbuf0, aggr_sh.at[idst_g.at[r0]], ss0, add=True)
            issue(gbase, isrc_g, r0 + 2, hbuf0, gbuf0, sh0, sg0)
            wait(gbase, isrc_g, r1, hbuf1, gbuf1, sh1, sg1)
            if first_block:
                @pl.when(t > 0)
                def _():
                    wait_ss(mbuf1, ss1)
            else:
                wait_ss(mbuf1, ss1)
            mul(hbuf1, gbuf1, mbuf1)
            pltpu.async_copy(mbuf1, aggr_sh.at[idst_g.at[r1]], ss1, add=True)
            issue(gbase, isrc_g, r1 + 2, hbuf1, gbuf1, sh1, sg1)
            return carry

        lax.fori_loop(0, NPAIR_BLK, pair, 0)

    wait_ss(mbuf0, ss0)
    wait_ss(mbuf1, ss1)
    isrc_g, _ = staging[(NBLK - 1) % 2]
    wait((NBLK - 1) * G, isrc_g, G, hbuf0, gbuf0, sh0, sg0)
    wait((NBLK - 1) * G, isrc_g, G + 1, hbuf1, gbuf1, sh1, sg1)
    plsc.subcore_barrier()
    pltpu.sync_copy(
        aggr_sh.at[pl.ds(s * ROWS_PER_SUB, ROWS_PER_SUB)],
        out_hbm.at[c, pl.ds(s * ROWS_PER_SUB, ROWS_PER_SUB)],
    )


BE = 3200


def _pack_u16(q_half):
    r = q_half.reshape(-1, 4, 2, 16)
    return (r[:, :, 0, :] + r[:, :, 1, :] * 65536).reshape(-1, 4 * 16)


def _gate_body(rbf_ref, W1_ref, b1_ref, W2_ref, b2_ref, out_ref):
    g = _silu(jnp.dot(rbf_ref[...], W1_ref[...], preferred_element_type=jnp.float32)
              + b1_ref[...])
    gate = jax.nn.sigmoid(jnp.dot(g, W2_ref[...], preferred_element_type=jnp.float32)
                          + b2_ref[...])
    q = (gate * 65535.0 + 0.5).astype(jnp.int32)
    out_ref[0] = _pack_u16(q[:, :HALF])
    out_ref[1] = _pack_u16(q[:, HALF:])


def _gate_call(rbf, W1, b1, W2, b2):
    return pl.pallas_call(
        _gate_body,
        grid=(N_EDGES // BE,),
        in_specs=[
            pl.BlockSpec((BE, N_RBF), lambda i: (i, 0)),
            pl.BlockSpec((N_RBF, HIDDEN), lambda i: (0, 0)),
            pl.BlockSpec((1, HIDDEN), lambda i: (0, 0)),
            pl.BlockSpec((HIDDEN, HIDDEN), lambda i: (0, 0)),
            pl.BlockSpec((1, HIDDEN), lambda i: (0, 0)),
        ],
        out_specs=pl.BlockSpec((NC, BE, HALF // 2), lambda i: (0, i, 0)),
        out_shape=jax.ShapeDtypeStruct((NC, E_PAD, HALF // 2), jnp.int32),
    )(rbf, W1, b1.reshape(1, HIDDEN), W2, b2.reshape(1, HIDDEN))


BN = 2000


def _upd_body(h_ref, aggr_ref, U1a_ref, U1b_ref, c1_ref, U2_ref, c2_ref, out_ref):
    h = h_ref[...]
    acc = jnp.dot(h, U1a_ref[...], preferred_element_type=jnp.float32)
    acc += jnp.dot(aggr_ref[0], U1b_ref[0], preferred_element_type=jnp.float32)
    acc += jnp.dot(aggr_ref[1], U1b_ref[1], preferred_element_type=jnp.float32)
    u = _silu(acc + c1_ref[...])
    out_ref[...] = h + jnp.dot(u, U2_ref[...], preferred_element_type=jnp.float32) \
        + c2_ref[...]


def _upd_call(h, aggr2, U1a, U1b, c1, U2, c2):
    return pl.pallas_call(
        _upd_body,
        grid=(N_NODES // BN,),
        in_specs=[
            pl.BlockSpec((BN, HIDDEN), lambda i: (i, 0)),
            pl.BlockSpec((NC, BN, HALF), lambda i: (0, i, 0)),
            pl.BlockSpec((HIDDEN, HIDDEN), lambda i: (0, 0)),
            pl.BlockSpec((NC, HALF, HIDDEN), lambda i: (0, 0, 0)),
            pl.BlockSpec((1, HIDDEN), lambda i: (0, 0)),
            pl.BlockSpec((HIDDEN, HIDDEN), lambda i: (0, 0)),
            pl.BlockSpec((1, HIDDEN), lambda i: (0, 0)),
        ],
        out_specs=pl.BlockSpec((BN, HIDDEN), lambda i: (i, 0)),
        out_shape=jax.ShapeDtypeStruct((N_NODES, HIDDEN), jnp.float32),
    )(h, aggr2, U1a, U1b, c1.reshape(1, HIDDEN), U2, c2.reshape(1, HIDDEN))


def kernel(h, edge_index, rbf, W1, b1, W2, b2, U1, c1, U2, c2):
    src = edge_index[0]
    dst = edge_index[1]
    hcat = (h * (1.0 / 65535.0)).reshape(N_NODES, NC, HALF) \
        .transpose(1, 0, 2).reshape(NC * N_NODES, HALF)
    blk = jnp.arange(NBLK)[:, None] * G + jnp.arange(GP)[None, :]
    src_p = jnp.concatenate(
        [src.reshape(NS, NCHUNK, CHUNK),
         jnp.zeros((NS, 2, CHUNK), jnp.int32)], axis=1)
    src_b = src_p[:, blk, :]
    src2 = jnp.stack([src_b, src_b + N_NODES])
    dst_b = dst.reshape(NS, NBLK, G, CHUNK)
    zeros = jnp.zeros((ROWS_PER_SUB, HALF), jnp.float32)

    gate2 = _gate_call(rbf, W1, b1, W2, b2)
    aggr2 = _sc_fused(hcat, gate2, src2, dst_b, zeros)

    U1a = U1[:HIDDEN]
    U1b = U1[HIDDEN:].reshape(NC, HALF, HIDDEN)
    return _upd_call(h, aggr2, U1a, U1b, c1, U2, c2)


# speedup vs baseline: 3.7891x; 3.7891x over previous
import functools

import jax
import jax.numpy as jnp
from jax import lax
from jax.experimental import pallas as pl
from jax.experimental.pallas import tpu as pltpu
from jax.experimental.pallas import tpu_sc as plsc

N_NODES = 10000
N_EDGES = 160000
HIDDEN = 256
HALF = 128
N_RBF = 16

NC = 2
NS = 16
CHUNK = 40
EDGES_PER_SUB = N_EDGES // NS
NCHUNK = EDGES_PER_SUB // CHUNK
G = 10
GP = G + 2
NBLK = NCHUNK // G
NPAIR_BLK = G // 2
N_PAD = 10112
ROWS_PER_SUB = N_PAD // NS
E_PAD = (NS - 1) * EDGES_PER_SUB + (NCHUNK + 2) * CHUNK


def _silu(x):
    return x * jax.nn.sigmoid(x)


_sc_mesh = plsc.VectorSubcoreMesh(core_axis_name="c", subcore_axis_name="s")


@functools.partial(
    pl.kernel,
    out_type=jax.ShapeDtypeStruct((NC, N_PAD, HALF), jnp.float32),
    scratch_types=[
        pltpu.VMEM((GP, CHUNK), jnp.int32),
        pltpu.VMEM((GP, CHUNK), jnp.int32),
        pltpu.VMEM((G, CHUNK), jnp.int32),
        pltpu.VMEM((G, CHUNK), jnp.int32),
        pltpu.VMEM((CHUNK, HALF), jnp.float32),
        pltpu.VMEM((CHUNK, HALF), jnp.float32),
        pltpu.VMEM((CHUNK, HALF // 2), jnp.int32),
        pltpu.VMEM((CHUNK, HALF // 2), jnp.int32),
        pltpu.VMEM((CHUNK, HALF), jnp.float32),
        pltpu.VMEM((CHUNK, HALF), jnp.float32),
        pltpu.VMEM_SHARED((N_PAD, HALF), jnp.float32),
        pltpu.SemaphoreType.DMA,
        pltpu.SemaphoreType.DMA,
        pltpu.SemaphoreType.DMA,
        pltpu.SemaphoreType.DMA,
        pltpu.SemaphoreType.DMA,
        pltpu.SemaphoreType.DMA,
    ],
    mesh=_sc_mesh,
)
def _sc_fused(hcat_hbm, gate_hbm, src_hbm, dst_hbm, zeros_hbm, out_hbm,
              isrcA, isrcB, idstA, idstB, hbuf0, hbuf1, gbuf0, gbuf1, mbuf0,
              mbuf1, aggr_sh, sh0, sh1, sg0, sg1, ss0, ss1):
    c = lax.axis_index("c")
    s = lax.axis_index("s")
    pltpu.sync_copy(zeros_hbm, aggr_sh.at[pl.ds(s * ROWS_PER_SUB, ROWS_PER_SUB)])
    plsc.subcore_barrier()
    ebase = s * EDGES_PER_SUB
    staging = [(isrcA, idstA), (isrcB, idstB)]

    def issue(gbase, isrc_g, r, hbuf, gbuf, sh, sg):
        pltpu.async_copy(hcat_hbm.at[isrc_g.at[r]], hbuf, sh)
        pltpu.async_copy(gate_hbm.at[c, pl.ds(ebase + (gbase + r) * CHUNK, CHUNK)],
                         gbuf, sg)

    def wait(gbase, isrc_g, r, hbuf, gbuf, sh, sg):
        pltpu.make_async_copy(hcat_hbm.at[isrc_g.at[r]], hbuf, sh).wait()
        pltpu.make_async_copy(
            gate_hbm.at[c, pl.ds(ebase + (gbase + r) * CHUNK, CHUNK)],
            gbuf, sg).wait()

    def wait_ss(mbuf, sem):
        pltpu.make_async_copy(mbuf, aggr_sh.at[pl.ds(0, CHUNK)], sem).wait()

    def mul(hbuf, gbuf, mbuf):
        def row(r, carry):
            for k in range(HALF // 32):
                g32 = gbuf[r, pl.ds(k * 16, 16)]
                lo = (g32 & 0xFFFF).astype(jnp.float32)
                hi = lax.shift_right_logical(g32, 16).astype(jnp.float32)
                sl0 = pl.ds(k * 16, 16)
                sl1 = pl.ds(64 + k * 16, 16)
                mbuf[r, sl0] = lo * hbuf[r, sl0]
                mbuf[r, sl1] = hi * hbuf[r, sl1]
            return carry
        lax.fori_loop(0, CHUNK, row, 0)

    pltpu.sync_copy(src_hbm.at[c, s, 0], isrcA)
    pltpu.sync_copy(dst_hbm.at[s, 0], idstA)
    issue(0, isrcA, 0, hbuf0, gbuf0, sh0, sg0)
    issue(0, isrcA, 1, hbuf1, gbuf1, sh1, sg1)

    for g in range(NBLK):
        isrc_g, idst_g = staging[g % 2]
        if g > 0:
            pltpu.sync_copy(src_hbm.at[c, s, g], isrc_g)
            pltpu.sync_copy(dst_hbm.at[s, g], idst_g)
        gbase = g * G
        first_block = g == 0

        def pair(t, carry, isrc_g=isrc_g, idst_g=idst_g, gbase=gbase,
                 first_block=first_block):
            r0 = 2 * t
            r1 = r0 + 1
            wait(gbase, isrc_g, r0, hbuf0, gbuf0, sh0, sg0)
            if first_block:
                @pl.when(t > 0)
                def _():
                    wait_ss(mbuf0, ss0)
            else:
                wait_ss(mbuf0, ss0)
            mul(hbuf0, gbuf0, mbuf0)
            pltpu.async_copy(mbuf0, aggr_sh.at[idst_g.at[r0]], ss0, add=True)
            issue(gbase, isrc_g, r0 + 2, hbuf0, gbuf0, sh0, sg0)
            wait(gbase, isrc_g, r1, hbuf1, gbuf1, sh1, sg1)
            if first_block:
                @pl.when(t > 0)
                def _():
                    wait_ss(mbuf1, ss1)
            else:
                wait_ss(mbuf1, ss1)
            mul(hbuf1, gbuf1, mbuf1)
            pltpu.async_copy(mbuf1, aggr_sh.at[idst_g.at[r1]], ss1, add=True)
            issue(gbase, isrc_g, r1 + 2, hbuf1, gbuf1, sh1, sg1)
            return carry

        lax.fori_loop(0, NPAIR_BLK, pair, 0)

    wait_ss(mbuf0, ss0)
    wait_ss(mbuf1, ss1)
    isrc_g, _ = staging[(NBLK - 1) % 2]
    wait((NBLK - 1) * G, isrc_g, G, hbuf0, gbuf0, sh0, sg0)
    wait((NBLK - 1) * G, isrc_g, G + 1, hbuf1, gbuf1, sh1, sg1)
    plsc.subcore_barrier()
    pltpu.sync_copy(
        aggr_sh.at[pl.ds(s * ROWS_PER_SUB, ROWS_PER_SUB)],
        out_hbm.at[c, pl.ds(s * ROWS_PER_SUB, ROWS_PER_SUB)],
    )


BE = 3200


def _pack_u16(q_half):
    return q_half[:, :64] + q_half[:, 64:] * 65536


def _gate_body(rbf_ref, W1_ref, b1_ref, W2_ref, b2_ref, out_ref):
    g = _silu(jnp.dot(rbf_ref[...], W1_ref[...], preferred_element_type=jnp.float32)
              + b1_ref[...])
    gate = jax.nn.sigmoid(jnp.dot(g, W2_ref[...], preferred_element_type=jnp.float32)
                          + b2_ref[...])
    q = (gate * 65535.0 + 0.5).astype(jnp.int32)
    out_ref[0] = _pack_u16(q[:, :HALF])
    out_ref[1] = _pack_u16(q[:, HALF:])


def _gate_call(rbf, W1, b1, W2, b2):
    return pl.pallas_call(
        _gate_body,
        grid=(N_EDGES // BE,),
        in_specs=[
            pl.BlockSpec((BE, N_RBF), lambda i: (i, 0)),
            pl.BlockSpec((N_RBF, HIDDEN), lambda i: (0, 0)),
            pl.BlockSpec((1, HIDDEN), lambda i: (0, 0)),
            pl.BlockSpec((HIDDEN, HIDDEN), lambda i: (0, 0)),
            pl.BlockSpec((1, HIDDEN), lambda i: (0, 0)),
        ],
        out_specs=pl.BlockSpec((NC, BE, HALF // 2), lambda i: (0, i, 0)),
        out_shape=jax.ShapeDtypeStruct((NC, E_PAD, HALF // 2), jnp.int32),
    )(rbf, W1, b1.reshape(1, HIDDEN), W2, b2.reshape(1, HIDDEN))


BN = 2000


def _upd_body(h_ref, aggr_ref, U1a_ref, U1b_ref, c1_ref, U2_ref, c2_ref, out_ref):
    h = h_ref[...]
    acc = jnp.dot(h, U1a_ref[...], preferred_element_type=jnp.float32)
    acc += jnp.dot(aggr_ref[0], U1b_ref[0], preferred_element_type=jnp.float32)
    acc += jnp.dot(aggr_ref[1], U1b_ref[1], preferred_element_type=jnp.float32)
    u = _silu(acc + c1_ref[...])
    out_ref[...] = h + jnp.dot(u, U2_ref[...], preferred_element_type=jnp.float32) \
        + c2_ref[...]


def _upd_call(h, aggr2, U1a, U1b, c1, U2, c2):
    return pl.pallas_call(
        _upd_body,
        grid=(N_NODES // BN,),
        in_specs=[
            pl.BlockSpec((BN, HIDDEN), lambda i: (i, 0)),
            pl.BlockSpec((NC, BN, HALF), lambda i: (0, i, 0)),
            pl.BlockSpec((HIDDEN, HIDDEN), lambda i: (0, 0)),
            pl.BlockSpec((NC, HALF, HIDDEN), lambda i: (0, 0, 0)),
            pl.BlockSpec((1, HIDDEN), lambda i: (0, 0)),
            pl.BlockSpec((HIDDEN, HIDDEN), lambda i: (0, 0)),
            pl.BlockSpec((1, HIDDEN), lambda i: (0, 0)),
        ],
        out_specs=pl.BlockSpec((BN, HIDDEN), lambda i: (i, 0)),
        out_shape=jax.ShapeDtypeStruct((N_NODES, HIDDEN), jnp.float32),
    )(h, aggr2, U1a, U1b, c1.reshape(1, HIDDEN), U2, c2.reshape(1, HIDDEN))


def kernel(h, edge_index, rbf, W1, b1, W2, b2, U1, c1, U2, c2):
    src = edge_index[0]
    dst = edge_index[1]
    hcat = (h * (1.0 / 65535.0)).reshape(N_NODES, NC, HALF) \
        .transpose(1, 0, 2).reshape(NC * N_NODES, HALF)
    blk = jnp.arange(NBLK)[:, None] * G + jnp.arange(GP)[None, :]
    src_p = jnp.concatenate(
        [src.reshape(NS, NCHUNK, CHUNK),
         jnp.zeros((NS, 2, CHUNK), jnp.int32)], axis=1)
    src_b = src_p[:, blk, :]
    src2 = jnp.stack([src_b, src_b + N_NODES])
    dst_b = dst.reshape(NS, NBLK, G, CHUNK)
    zeros = jnp.zeros((ROWS_PER_SUB, HALF), jnp.float32)

    gate2 = _gate_call(rbf, W1, b1, W2, b2)
    aggr2 = _sc_fused(hcat, gate2, src2, dst_b, zeros)

    U1a = U1[:HIDDEN]
    U1b = U1[HIDDEN:].reshape(NC, HALF, HIDDEN)
    return _upd_call(h, aggr2, U1a, U1b, c1, U2, c2)


# speedup vs baseline: 3.9843x; 1.0515x over previous
import functools

import jax
import jax.numpy as jnp
from jax import lax
from jax.experimental import pallas as pl
from jax.experimental.pallas import tpu as pltpu
from jax.experimental.pallas import tpu_sc as plsc

N_NODES = 10000
N_EDGES = 160000
HIDDEN = 256
HALF = 128
N_RBF = 16

NC = 2
NS = 16
CHUNK = 80
EDGES_PER_SUB = N_EDGES // NS
NCHUNK = EDGES_PER_SUB // CHUNK
N_PAD = 10240
ROWS_PER_SUB = N_PAD // NS


def _silu(x):
    return x * jax.nn.sigmoid(x)


_sc_mesh = plsc.VectorSubcoreMesh(core_axis_name="c", subcore_axis_name="s")


NPAIR = (NCHUNK - 1) // 2


@functools.partial(
    pl.kernel,
    out_type=jax.ShapeDtypeStruct((NC, N_EDGES, HALF), jnp.float32),
    scratch_types=[
        pltpu.VMEM((NCHUNK, CHUNK), jnp.int32),
        pltpu.VMEM((CHUNK, HALF), jnp.float32),
        pltpu.VMEM((CHUNK, HALF), jnp.float32),
        pltpu.SemaphoreType.DMA,
        pltpu.SemaphoreType.DMA,
    ],
    mesh=_sc_mesh,
)
def _sc_gather(hcat_hbm, src2_hbm, out_hbm, idx_v, buf0, buf1, sem0, sem1):
    c = lax.axis_index("c")
    s = lax.axis_index("s")
    pltpu.sync_copy(src2_hbm.at[c, s], idx_v)
    ebase = s * EDGES_PER_SUB
    pltpu.async_copy(hcat_hbm.at[idx_v.at[0]], buf0, sem0)

    def body(t, carry):
        j0 = 2 * t
        pltpu.async_copy(hcat_hbm.at[idx_v.at[j0 + 1]], buf1, sem1)
        pltpu.make_async_copy(hcat_hbm.at[idx_v.at[j0]], buf0, sem0).wait()
        pltpu.sync_copy(buf0, out_hbm.at[c, pl.ds(ebase + j0 * CHUNK, CHUNK)])
        pltpu.async_copy(hcat_hbm.at[idx_v.at[j0 + 2]], buf0, sem0)
        pltpu.make_async_copy(hcat_hbm.at[idx_v.at[j0 + 1]], buf1, sem1).wait()
        pltpu.sync_copy(buf1, out_hbm.at[c, pl.ds(ebase + (j0 + 1) * CHUNK, CHUNK)])
        return carry

    lax.fori_loop(0, NPAIR, body, 0)
    j_last = NCHUNK - 1
    pltpu.make_async_copy(hcat_hbm.at[idx_v.at[j_last]], buf0, sem0).wait()
    pltpu.sync_copy(buf0, out_hbm.at[c, pl.ds(ebase + j_last * CHUNK, CHUNK)])


@functools.partial(
    pl.kernel,
    out_type=jax.ShapeDtypeStruct((NC, N_PAD, HALF), jnp.float32),
    scratch_types=[
        pltpu.VMEM((NCHUNK, CHUNK), jnp.int32),
        pltpu.VMEM((CHUNK, HALF), jnp.float32),
        pltpu.VMEM((CHUNK, HALF), jnp.float32),
        pltpu.VMEM_SHARED((N_PAD, HALF), jnp.float32),
        pltpu.SemaphoreType.DMA,
        pltpu.SemaphoreType.DMA,
    ],
    mesh=_sc_mesh,
)
def _sc_scatter(msg_hbm, dst_hbm, zeros_hbm, out_hbm, idx_v, buf0, buf1, aggr_sh,
                sem0, sem1):
    c = lax.axis_index("c")
    s = lax.axis_index("s")
    pltpu.sync_copy(dst_hbm.at[s], idx_v)
    pltpu.sync_copy(zeros_hbm, aggr_sh.at[pl.ds(s * ROWS_PER_SUB, ROWS_PER_SUB)])
    plsc.subcore_barrier()
    ebase = s * EDGES_PER_SUB
    pltpu.async_copy(msg_hbm.at[c, pl.ds(ebase, CHUNK)], buf0, sem0)

    def body(t, carry):
        j0 = 2 * t
        pltpu.async_copy(msg_hbm.at[c, pl.ds(ebase + (j0 + 1) * CHUNK, CHUNK)],
                         buf1, sem1)
        pltpu.make_async_copy(msg_hbm.at[c, pl.ds(ebase + j0 * CHUNK, CHUNK)],
                              buf0, sem0).wait()
        pltpu.sync_copy(buf0, aggr_sh.at[idx_v.at[j0]], add=True)
        pltpu.async_copy(msg_hbm.at[c, pl.ds(ebase + (j0 + 2) * CHUNK, CHUNK)],
                         buf0, sem0)
        pltpu.make_async_copy(msg_hbm.at[c, pl.ds(ebase + (j0 + 1) * CHUNK, CHUNK)],
                              buf1, sem1).wait()
        pltpu.sync_copy(buf1, aggr_sh.at[idx_v.at[j0 + 1]], add=True)
        return carry

    lax.fori_loop(0, NPAIR, body, 0)
    j_last = NCHUNK - 1
    pltpu.make_async_copy(msg_hbm.at[c, pl.ds(ebase + j_last * CHUNK, CHUNK)],
                          buf0, sem0).wait()
    pltpu.sync_copy(buf0, aggr_sh.at[idx_v.at[j_last]], add=True)
    plsc.subcore_barrier()
    pltpu.sync_copy(
        aggr_sh.at[pl.ds(s * ROWS_PER_SUB, ROWS_PER_SUB)],
        out_hbm.at[c, pl.ds(s * ROWS_PER_SUB, ROWS_PER_SUB)],
    )


BE = 3200


def _msg_body(rbf_ref, hsrc_ref, W1_ref, b1_ref, W2_ref, b2_ref, out_ref):
    g = _silu(jnp.dot(rbf_ref[...], W1_ref[...], preferred_element_type=jnp.float32)
              + b1_ref[...])
    gate = jax.nn.sigmoid(jnp.dot(g, W2_ref[...], preferred_element_type=jnp.float32)
                          + b2_ref[...])
    out_ref[0] = gate[:, :HALF] * hsrc_ref[0]
    out_ref[1] = gate[:, HALF:] * hsrc_ref[1]


def _msg_call(rbf, hsrc2, W1, b1, W2, b2):
    return pl.pallas_call(
        _msg_body,
        grid=(N_EDGES // BE,),
        in_specs=[
            pl.BlockSpec((BE, N_RBF), lambda i: (i, 0)),
            pl.BlockSpec((NC, BE, HALF), lambda i: (0, i, 0)),
            pl.BlockSpec((N_RBF, HIDDEN), lambda i: (0, 0)),
            pl.BlockSpec((1, HIDDEN), lambda i: (0, 0)),
            pl.BlockSpec((HIDDEN, HIDDEN), lambda i: (0, 0)),
            pl.BlockSpec((1, HIDDEN), lambda i: (0, 0)),
        ],
        out_specs=pl.BlockSpec((NC, BE, HALF), lambda i: (0, i, 0)),
        out_shape=jax.ShapeDtypeStruct((NC, N_EDGES, HALF), jnp.float32),
    )(rbf, hsrc2, W1, b1.reshape(1, HIDDEN), W2, b2.reshape(1, HIDDEN))


BN = 2000


def _upd_body(h_ref, aggr_ref, U1a_ref, U1b_ref, c1_ref, U2_ref, c2_ref, out_ref):
    h = h_ref[...]
    acc = jnp.dot(h, U1a_ref[...], preferred_element_type=jnp.float32)
    acc += jnp.dot(aggr_ref[0], U1b_ref[0], preferred_element_type=jnp.float32)
    acc += jnp.dot(aggr_ref[1], U1b_ref[1], preferred_element_type=jnp.float32)
    u = _silu(acc + c1_ref[...])
    out_ref[...] = h + jnp.dot(u, U2_ref[...], preferred_element_type=jnp.float32) \
        + c2_ref[...]


def _upd_call(h, aggr2, U1a, U1b, c1, U2, c2):
    return pl.pallas_call(
        _upd_body,
        grid=(N_NODES // BN,),
        in_specs=[
            pl.BlockSpec((BN, HIDDEN), lambda i: (i, 0)),
            pl.BlockSpec((NC, BN, HALF), lambda i: (0, i, 0)),
            pl.BlockSpec((HIDDEN, HIDDEN), lambda i: (0, 0)),
            pl.BlockSpec((NC, HALF, HIDDEN), lambda i: (0, 0, 0)),
            pl.BlockSpec((1, HIDDEN), lambda i: (0, 0)),
            pl.BlockSpec((HIDDEN, HIDDEN), lambda i: (0, 0)),
            pl.BlockSpec((1, HIDDEN), lambda i: (0, 0)),
        ],
        out_specs=pl.BlockSpec((BN, HIDDEN), lambda i: (i, 0)),
        out_shape=jax.ShapeDtypeStruct((N_NODES, HIDDEN), jnp.float32),
    )(h, aggr2, U1a, U1b, c1.reshape(1, HIDDEN), U2, c2.reshape(1, HIDDEN))


def kernel(h, edge_index, rbf, W1, b1, W2, b2, U1, c1, U2, c2):
    src = edge_index[0]
    dst = edge_index[1]
    hcat = h.reshape(N_NODES, NC, HALF).transpose(1, 0, 2).reshape(NC * N_NODES, HALF)
    src_r = src.reshape(NS, NCHUNK, CHUNK)
    src2 = jnp.stack([src_r, src_r + N_NODES])
    dst_r = dst.reshape(NS, NCHUNK, CHUNK)
    zeros = jnp.zeros((ROWS_PER_SUB, HALF), jnp.float32)

    hsrc2 = _sc_gather(hcat, src2)
    msg2 = _msg_call(rbf, hsrc2, W1, b1, W2, b2)
    aggr2 = _sc_scatter(msg2, dst_r, zeros)

    U1a = U1[:HIDDEN]
    U1b = U1[HIDDEN:].reshape(NC, HALF, HIDDEN)
    return _upd_call(h, aggr2, U1a, U1b, c1, U2, c2)
